# Initial kernel scaffold; baseline (speedup 1.0000x reference)
import jax
import jax.numpy as jnp
from jax.experimental import pallas as pl


def kernel(inputs):
    raise NotImplementedError("write your pallas kernel here")


# speedup vs baseline: 9.6068x; 9.6068x over previous
import functools

import jax
import jax.numpy as jnp
import numpy as np
from jax import lax
from jax.experimental import pallas as pl
from jax.experimental.pallas import tpu as pltpu
from jax.experimental.pallas import tpu_sc as plsc

_B = 16384
_C = 1000
_K = 26

_INFO = plsc.get_sparse_core_info()
_NC = _INFO.num_cores
_NS = _INFO.num_subcores
_L = _INFO.num_lanes
_NW = _NC * _NS
_ROWS = _B // _NW
_CHUNK = 64
_NCHUNK = _ROWS // _CHUNK
_GROUPS = _CHUNK * _K // _L

_ROW_OFF = ((np.arange(_CHUNK * _K) // _K) * _C).astype(np.int32)

_mesh = plsc.VectorSubcoreMesh(core_axis_name="c", subcore_axis_name="s")


@functools.partial(
    pl.kernel,
    mesh=_mesh,
    compiler_params=pltpu.CompilerParams(needs_layout_passes=False),
    out_type=jax.ShapeDtypeStruct((_B * _C,), jnp.float32),
    scratch_types=[
        pltpu.VMEM((_ROWS * _K,), jnp.int32),
        pltpu.VMEM((_CHUNK * _C,), jnp.float32),
        pltpu.VMEM((_CHUNK * _K,), jnp.int32),
    ],
)
def _multihot(idx_hbm, rowoff_hbm, out_hbm, idx_v, buf_v, tab_v):
    wid = lax.axis_index("s") * _NC + lax.axis_index("c")
    rbase = wid * _ROWS

    pltpu.sync_copy(idx_hbm.at[pl.ds(rbase * _K, _ROWS * _K)], idx_v)
    pltpu.sync_copy(rowoff_hbm, tab_v)

    zeros = jnp.zeros((_L,), jnp.float32)
    ones = jnp.full((_L,), 1.0, jnp.float32)

    def _clear(i, carry):
        buf_v[pl.ds(i * _L, _L)] = zeros
        return carry

    lax.fori_loop(0, _CHUNK * _C // _L, _clear, 0)

    def _scatter(c, val):
        base = c * (_CHUNK * _K)

        def g(i, carry):
            col = idx_v[pl.ds(base + i * _L, _L)]
            roff = tab_v[pl.ds(i * _L, _L)]
            plsc.store_scatter(buf_v, [roff + col], val)
            return carry

        lax.fori_loop(0, _GROUPS, g, 0)

    def _chunk(c, carry):
        _scatter(c, ones)
        pltpu.sync_copy(
            buf_v, out_hbm.at[pl.ds((rbase + c * _CHUNK) * _C, _CHUNK * _C)]
        )
        _scatter(c, zeros)
        return carry

    lax.fori_loop(0, _NCHUNK, _chunk, 0)


def kernel(inputs):
    flat = inputs.reshape(_B * _K)
    out = _multihot(flat, jnp.asarray(_ROW_OFF))
    return out.reshape(_B, _C)


# speedup vs baseline: 11.0503x; 1.1503x over previous
import functools

import jax
import jax.numpy as jnp
import numpy as np
from jax import lax
from jax.experimental import pallas as pl
from jax.experimental.pallas import tpu as pltpu
from jax.experimental.pallas import tpu_sc as plsc

_B = 16384
_C = 1000
_K = 26

_INFO = plsc.get_sparse_core_info()
_NC = _INFO.num_cores
_NS = _INFO.num_subcores
_L = _INFO.num_lanes
_NW = _NC * _NS
_ROWS = _B // _NW
_CHUNK = 32
_NCHUNK = _ROWS // _CHUNK
_GROUPS = _CHUNK * _K // _L
_UN = 4
_CLEAR_UN = 8

_ROW_OFF = ((np.arange(_CHUNK * _K) // _K) * _C).astype(np.int32)

_mesh = plsc.VectorSubcoreMesh(core_axis_name="c", subcore_axis_name="s")


@functools.partial(
    pl.kernel,
    mesh=_mesh,
    compiler_params=pltpu.CompilerParams(needs_layout_passes=False),
    out_type=jax.ShapeDtypeStruct((_B * _C,), jnp.float32),
    scratch_types=[
        pltpu.VMEM((_ROWS * _K,), jnp.int32),
        pltpu.VMEM((_CHUNK * _C,), jnp.float32),
        pltpu.VMEM((_CHUNK * _C,), jnp.float32),
        pltpu.VMEM((_CHUNK * _K,), jnp.int32),
        pltpu.SemaphoreType.DMA,
        pltpu.SemaphoreType.DMA,
    ],
)
def _multihot(idx_hbm, rowoff_hbm, out_hbm, idx_v, buf0, buf1, tab_v, sem0, sem1):
    wid = lax.axis_index("s") * _NC + lax.axis_index("c")
    rbase = wid * _ROWS

    pltpu.sync_copy(idx_hbm.at[pl.ds(rbase * _K, _ROWS * _K)], idx_v)
    pltpu.sync_copy(rowoff_hbm, tab_v)

    bufs = (buf0, buf1)
    sems = (sem0, sem1)
    zeros = jnp.zeros((_L,), jnp.float32)
    ones = jnp.full((_L,), 1.0, jnp.float32)

    def _clear(buf):
        def body(i, carry):
            for u in range(_CLEAR_UN):
                buf[pl.ds((i * _CLEAR_UN + u) * _L, _L)] = zeros
            return carry

        lax.fori_loop(0, _CHUNK * _C // (_L * _CLEAR_UN), body, 0)

    def _scatter(buf, c, val):
        base = c * (_CHUNK * _K)

        def body(i, carry):
            p = i * (_UN * _L)
            for u in range(_UN):
                col = idx_v[pl.ds(base + p + u * _L, _L)]
                roff = tab_v[pl.ds(p + u * _L, _L)]
                plsc.store_scatter(buf, [roff + col], val)
            return carry

        lax.fori_loop(0, _GROUPS // _UN, body, 0)

    def _stream(buf, c, sem):
        return pltpu.async_copy(
            buf, out_hbm.at[pl.ds((rbase + c * _CHUNK) * _C, _CHUNK * _C)], sem
        )

    copies = [None] * _NCHUNK
    _clear(buf0)
    _clear(buf1)
    for c in range(_NCHUNK):
        b = c % 2
        if c >= 2:
            copies[c - 2].wait()
            _scatter(bufs[b], c - 2, zeros)
        _scatter(bufs[b], c, ones)
        copies[c] = _stream(bufs[b], c, sems[b])
    copies[_NCHUNK - 2].wait()
    copies[_NCHUNK - 1].wait()


def kernel(inputs):
    flat = inputs.reshape(_B * _K)
    out = _multihot(flat, jnp.asarray(_ROW_OFF))
    return out.reshape(_B, _C)


# speedup vs baseline: 16.8386x; 1.5238x over previous
import functools

import jax
import jax.numpy as jnp
import numpy as np
from jax import lax
from jax.experimental import pallas as pl
from jax.experimental.pallas import tpu as pltpu
from jax.experimental.pallas import tpu_sc as plsc

_B = 16384
_C = 1000
_K = 26

_INFO = plsc.get_sparse_core_info()
_NC = _INFO.num_cores
_NS = _INFO.num_subcores
_L = _INFO.num_lanes
_NW = _NC * _NS
_ROWS = _B // _NW
_CHUNK = 32
_NCHUNK = _ROWS // _CHUNK
_GROUPS = _CHUNK * _K // _L
_UN = 4
_CLEAR_UN = 8

_ROW_TAB = (np.arange(_CHUNK * _K) // _K).astype(np.int32)

_mesh = plsc.VectorSubcoreMesh(core_axis_name="c", subcore_axis_name="s")


@functools.partial(
    pl.kernel,
    mesh=_mesh,
    compiler_params=pltpu.CompilerParams(needs_layout_passes=False),
    out_type=jax.ShapeDtypeStruct((_B, _C), jnp.float32),
    scratch_types=[
        pltpu.VMEM((_ROWS * _K,), jnp.int32),
        pltpu.VMEM((_CHUNK, _C), jnp.float32),
        pltpu.VMEM((_CHUNK, _C), jnp.float32),
        pltpu.VMEM((_CHUNK * _K,), jnp.int32),
        pltpu.SemaphoreType.DMA,
        pltpu.SemaphoreType.DMA,
    ],
)
def _multihot(idx_hbm, rowtab_hbm, out_hbm, idx_v, buf0, buf1, tab_v, sem0, sem1):
    wid = lax.axis_index("s") * _NC + lax.axis_index("c")
    rbase = wid * _ROWS

    pltpu.sync_copy(idx_hbm.at[pl.ds(rbase * _K, _ROWS * _K)], idx_v)
    pltpu.sync_copy(rowtab_hbm, tab_v)

    zeros = jnp.zeros((_L,), jnp.float32)
    ones = jnp.full((_L,), 1.0, jnp.float32)

    def _clear(buf):
        def row_body(r, carry):
            def col_body(j, carry2):
                for u in range(_CLEAR_UN):
                    buf[r, pl.ds((j * _CLEAR_UN + u) * _L, _L)] = zeros
                return carry2

            lax.fori_loop(0, (_C // _L) // _CLEAR_UN, col_body, 0)
            for u in range(_C // _L - (_C // _L) // _CLEAR_UN * _CLEAR_UN):
                buf[r, pl.ds(((_C // _L) // _CLEAR_UN * _CLEAR_UN + u) * _L, _L)] = zeros
            buf[r, pl.ds(_C - _L, _L)] = zeros
            return carry

        lax.fori_loop(0, _CHUNK, row_body, 0)

    def _scatter(buf, c, val):
        base = c * (_CHUNK * _K)

        def body(i, carry):
            p = i * (_UN * _L)
            for u in range(_UN):
                col = idx_v[pl.ds(base + p + u * _L, _L)]
                row = tab_v[pl.ds(p + u * _L, _L)]
                plsc.store_scatter(buf, [row, col], val)
            return carry

        lax.fori_loop(0, _GROUPS // _UN, body, 0)

    def _stream(buf, c, sem):
        return pltpu.async_copy(
            buf, out_hbm.at[pl.ds(rbase + c * _CHUNK, _CHUNK), :], sem
        )

    bufs = (buf0, buf1)
    sems = (sem0, sem1)
    copies = [None] * _NCHUNK
    _clear(buf0)
    _clear(buf1)
    for c in range(_NCHUNK):
        b = c % 2
        if c >= 2:
            copies[c - 2].wait()
            _scatter(bufs[b], c - 2, zeros)
        _scatter(bufs[b], c, ones)
        copies[c] = _stream(bufs[b], c, sems[b])
    copies[_NCHUNK - 2].wait()
    copies[_NCHUNK - 1].wait()


def kernel(inputs):
    flat = inputs.reshape(_B * _K)
    return _multihot(flat, jnp.asarray(_ROW_TAB))
